# baseline (speedup 1.0000x reference)
import jax
import jax.numpy as jnp
from jax.experimental import pallas as pl
from jax.experimental.pallas import tpu as pltpu

_B, _S1, _S2, _S3, _H = 16, 256, 64, 128, 64
_T = 64


def _leaky(x):
    return jnp.maximum(x, 0.01 * x)


def _fused_kernel(x_ref, bias_ref, w1_ref, w2a_ref, w2b_ref, w3a_ref,
                  w3b_ref, b1_ref, b2_ref, b3_ref, out_ref):
    t, s2, s3 = x_ref.shape
    th = t // 2
    rows = th * s2
    x = x_ref[...].reshape(t * s2, s3)
    xp = jnp.concatenate([x[:rows], x[rows:]], axis=1)

    bias = bias_ref[...]
    bias3 = jnp.concatenate(
        [jnp.broadcast_to(bias[:th], (th, s2, _H)),
         jnp.broadcast_to(bias[th:], (th, s2, _H))], axis=2)

    h1 = _leaky(jnp.dot(xp, w1_ref[...],
                        preferred_element_type=jnp.float32) + b1_ref[...])
    pool1 = jnp.max(h1.reshape(th, s2, 2 * _H) + bias3, axis=1)
    anym = pool1 != -jnp.inf
    pool1 = jnp.where(anym, pool1, 0.0)

    c1 = jnp.dot(pool1, w2b_ref[...],
                 preferred_element_type=jnp.float32) + b2_ref[...]
    h2 = _leaky((jnp.dot(h1, w2a_ref[...], preferred_element_type=jnp.float32)
                 .reshape(th, s2, 2 * _H)) + c1[:, None, :])
    pool2 = jnp.max(h2 + bias3, axis=1)
    pool2 = jnp.where(anym, pool2, 0.0)

    c2 = jnp.dot(pool2, w3b_ref[...],
                 preferred_element_type=jnp.float32) + b3_ref[...]
    z3 = jnp.dot(h2.reshape(rows, 2 * _H), w3a_ref[...],
                 preferred_element_type=jnp.float32).reshape(th, s2, 2 * _H)
    pool3 = _leaky(jnp.max(z3 + bias3, axis=1) + c2)
    outp = jnp.where(anym, pool3, 0.0)
    out_ref[...] = jnp.concatenate([outp[:, :_H], outp[:, _H:]], axis=0)


def _blockdiag(w):
    h, wd = w.shape
    z = jnp.zeros_like(w)
    return jnp.concatenate([jnp.concatenate([w, z], axis=1),
                            jnp.concatenate([z, w], axis=1)], axis=0)


def kernel(input_var, input_mask, size1, size2, W1, b1, W2, b2, W3, b3):
    p = _B * _S1
    h = _H
    x = input_var.reshape(p, _S2, _S3)
    bias = jnp.where(input_mask, 0.0, -jnp.inf).astype(jnp.float32)
    bias = bias.reshape(p, _S2, 1)
    w1 = _blockdiag(W1.T)
    w2a = _blockdiag(W2[:, :h].T)
    w2b = _blockdiag(W2[:, h:].T)
    w3a = _blockdiag(W3[:, :h].T)
    w3b = _blockdiag(W3[:, h:].T)
    b1p = jnp.tile(b1, 2)[None, :]
    b2p = jnp.tile(b2, 2)[None, :]
    b3p = jnp.tile(b3, 2)[None, :]

    grid = (p // _T,)
    out = pl.pallas_call(
        _fused_kernel,
        grid=grid,
        in_specs=[
            pl.BlockSpec((_T, _S2, _S3), lambda i: (i, 0, 0)),
            pl.BlockSpec((_T, _S2, 1), lambda i: (i, 0, 0)),
            pl.BlockSpec((2 * _S3, 2 * h), lambda i: (0, 0)),
            pl.BlockSpec((2 * h, 2 * h), lambda i: (0, 0)),
            pl.BlockSpec((2 * h, 2 * h), lambda i: (0, 0)),
            pl.BlockSpec((2 * h, 2 * h), lambda i: (0, 0)),
            pl.BlockSpec((2 * h, 2 * h), lambda i: (0, 0)),
            pl.BlockSpec((1, 2 * h), lambda i: (0, 0)),
            pl.BlockSpec((1, 2 * h), lambda i: (0, 0)),
            pl.BlockSpec((1, 2 * h), lambda i: (0, 0)),
        ],
        out_specs=pl.BlockSpec((_T, h), lambda i: (i, 0)),
        out_shape=jax.ShapeDtypeStruct((p, h), jnp.float32),
        compiler_params=pltpu.CompilerParams(
            dimension_semantics=("parallel",)),
    )(x, bias, w1, w2a, w2b, w3a, w3b, b1p, b2p, b3p)
    return out.reshape(_B, _S1, h)


# speedup vs baseline: 2.3108x; 2.3108x over previous
import jax
import jax.numpy as jnp
from jax.experimental import pallas as pl
from jax.experimental.pallas import tpu as pltpu

_B, _S1, _S2, _S3, _H = 16, 256, 64, 128, 64
_T = 64


def _leaky(x):
    return jnp.maximum(x, 0.01 * x)


def _fused_kernel(x_ref, bias_ref, w1_ref, w2a_ref, w2b_ref, w3a_ref,
                  w3b_ref, b1_ref, b2_ref, b3_ref, out_ref):
    t, s2, s3 = x_ref.shape
    th = t // 2
    rows = th * s2
    x = x_ref[...].reshape(t * s2, s3)
    xp = jnp.concatenate([x[:rows], x[rows:]], axis=1)

    bias = bias_ref[...]
    bias3 = jnp.concatenate(
        [jnp.broadcast_to(bias[:th, :, None], (th, s2, _H)),
         jnp.broadcast_to(bias[th:, :, None], (th, s2, _H))], axis=2)

    h1 = _leaky(jnp.dot(xp, w1_ref[...],
                        preferred_element_type=jnp.float32) + b1_ref[...])
    pool1 = jnp.max(h1.reshape(th, s2, 2 * _H) + bias3, axis=1)
    anym = pool1 != -jnp.inf
    pool1 = jnp.where(anym, pool1, 0.0)

    c1 = jnp.dot(pool1, w2b_ref[...],
                 preferred_element_type=jnp.float32) + b2_ref[...]
    h2 = _leaky((jnp.dot(h1, w2a_ref[...], preferred_element_type=jnp.float32)
                 .reshape(th, s2, 2 * _H)) + c1[:, None, :])
    pool2 = jnp.max(h2 + bias3, axis=1)
    pool2 = jnp.where(anym, pool2, 0.0)

    c2 = jnp.dot(pool2, w3b_ref[...],
                 preferred_element_type=jnp.float32) + b3_ref[...]
    z3 = jnp.dot(h2.reshape(rows, 2 * _H), w3a_ref[...],
                 preferred_element_type=jnp.float32).reshape(th, s2, 2 * _H)
    pool3 = _leaky(jnp.max(z3 + bias3, axis=1) + c2)
    outp = jnp.where(anym, pool3, 0.0)
    out_ref[...] = jnp.concatenate([outp[:, :_H], outp[:, _H:]], axis=0)


def _blockdiag(w):
    h, wd = w.shape
    z = jnp.zeros_like(w)
    return jnp.concatenate([jnp.concatenate([w, z], axis=1),
                            jnp.concatenate([z, w], axis=1)], axis=0)


def kernel(input_var, input_mask, size1, size2, W1, b1, W2, b2, W3, b3):
    p = _B * _S1
    h = _H
    x = input_var.reshape(p, _S2, _S3)
    bias = jnp.where(input_mask, 0.0, -jnp.inf).astype(jnp.float32)
    bias = bias.reshape(p, _S2)
    w1 = _blockdiag(W1.T)
    w2a = _blockdiag(W2[:, :h].T)
    w2b = _blockdiag(W2[:, h:].T)
    w3a = _blockdiag(W3[:, :h].T)
    w3b = _blockdiag(W3[:, h:].T)
    b1p = jnp.tile(b1, 2)[None, :]
    b2p = jnp.tile(b2, 2)[None, :]
    b3p = jnp.tile(b3, 2)[None, :]

    grid = (p // _T,)
    out = pl.pallas_call(
        _fused_kernel,
        grid=grid,
        in_specs=[
            pl.BlockSpec((_T, _S2, _S3), lambda i: (i, 0, 0)),
            pl.BlockSpec((_T, _S2), lambda i: (i, 0)),
            pl.BlockSpec((2 * _S3, 2 * h), lambda i: (0, 0)),
            pl.BlockSpec((2 * h, 2 * h), lambda i: (0, 0)),
            pl.BlockSpec((2 * h, 2 * h), lambda i: (0, 0)),
            pl.BlockSpec((2 * h, 2 * h), lambda i: (0, 0)),
            pl.BlockSpec((2 * h, 2 * h), lambda i: (0, 0)),
            pl.BlockSpec((1, 2 * h), lambda i: (0, 0)),
            pl.BlockSpec((1, 2 * h), lambda i: (0, 0)),
            pl.BlockSpec((1, 2 * h), lambda i: (0, 0)),
        ],
        out_specs=pl.BlockSpec((_T, h), lambda i: (i, 0)),
        out_shape=jax.ShapeDtypeStruct((p, h), jnp.float32),
        compiler_params=pltpu.CompilerParams(
            dimension_semantics=("parallel",)),
    )(x, bias, w1, w2a, w2b, w3a, w3b, b1p, b2p, b3p)
    return out.reshape(_B, _S1, h)


# speedup vs baseline: 2.6803x; 1.1599x over previous
import jax
import jax.numpy as jnp
from jax.experimental import pallas as pl
from jax.experimental.pallas import tpu as pltpu

_B, _S1, _S2, _S3, _H = 16, 256, 64, 128, 64
_T = 128


def _leaky(x):
    return jnp.maximum(x, 0.01 * x)


def _fused_kernel(x_ref, bias_ref, w1_ref, w2a_ref, w2b_ref, w3a_ref,
                  w3b_ref, b1_ref, b2_ref, b3_ref, out_ref):
    t, s2, s3 = x_ref.shape
    th = t // 2
    rows = th * s2
    x = x_ref[...].reshape(t * s2, s3)
    xp = jnp.concatenate([x[:rows], x[rows:]], axis=1)

    bias = bias_ref[...]
    bias3 = jnp.concatenate(
        [jnp.broadcast_to(bias[:th, :, None], (th, s2, _H)),
         jnp.broadcast_to(bias[th:, :, None], (th, s2, _H))], axis=2)

    h1 = _leaky(jnp.dot(xp, w1_ref[...],
                        preferred_element_type=jnp.float32) + b1_ref[...])
    pool1 = jnp.max(h1.reshape(th, s2, 2 * _H) + bias3, axis=1)
    anym = pool1 != -jnp.inf
    pool1 = jnp.where(anym, pool1, 0.0)

    c1 = jnp.dot(pool1, w2b_ref[...],
                 preferred_element_type=jnp.float32) + b2_ref[...]
    h2 = _leaky((jnp.dot(h1, w2a_ref[...], preferred_element_type=jnp.float32)
                 .reshape(th, s2, 2 * _H)) + c1[:, None, :])
    pool2 = jnp.max(h2 + bias3, axis=1)
    pool2 = jnp.where(anym, pool2, 0.0)

    c2 = jnp.dot(pool2, w3b_ref[...],
                 preferred_element_type=jnp.float32) + b3_ref[...]
    z3 = jnp.dot(h2.reshape(rows, 2 * _H), w3a_ref[...],
                 preferred_element_type=jnp.float32).reshape(th, s2, 2 * _H)
    pool3 = _leaky(jnp.max(z3 + bias3, axis=1) + c2)
    outp = jnp.where(anym, pool3, 0.0)
    out_ref[...] = jnp.concatenate([outp[:, :_H], outp[:, _H:]], axis=0)


def _blockdiag(w):
    h, wd = w.shape
    z = jnp.zeros_like(w)
    return jnp.concatenate([jnp.concatenate([w, z], axis=1),
                            jnp.concatenate([z, w], axis=1)], axis=0)


def kernel(input_var, input_mask, size1, size2, W1, b1, W2, b2, W3, b3):
    p = _B * _S1
    h = _H
    x = input_var.reshape(p, _S2, _S3)
    bias = jnp.where(input_mask, 0.0, -jnp.inf).astype(jnp.float32)
    bias = bias.reshape(p, _S2)
    w1 = _blockdiag(W1.T)
    w2a = _blockdiag(W2[:, :h].T)
    w2b = _blockdiag(W2[:, h:].T)
    w3a = _blockdiag(W3[:, :h].T)
    w3b = _blockdiag(W3[:, h:].T)
    b1p = jnp.tile(b1, 2)[None, :]
    b2p = jnp.tile(b2, 2)[None, :]
    b3p = jnp.tile(b3, 2)[None, :]

    grid = (p // _T,)
    out = pl.pallas_call(
        _fused_kernel,
        grid=grid,
        in_specs=[
            pl.BlockSpec((_T, _S2, _S3), lambda i: (i, 0, 0)),
            pl.BlockSpec((_T, _S2), lambda i: (i, 0)),
            pl.BlockSpec((2 * _S3, 2 * h), lambda i: (0, 0)),
            pl.BlockSpec((2 * h, 2 * h), lambda i: (0, 0)),
            pl.BlockSpec((2 * h, 2 * h), lambda i: (0, 0)),
            pl.BlockSpec((2 * h, 2 * h), lambda i: (0, 0)),
            pl.BlockSpec((2 * h, 2 * h), lambda i: (0, 0)),
            pl.BlockSpec((1, 2 * h), lambda i: (0, 0)),
            pl.BlockSpec((1, 2 * h), lambda i: (0, 0)),
            pl.BlockSpec((1, 2 * h), lambda i: (0, 0)),
        ],
        out_specs=pl.BlockSpec((_T, h), lambda i: (i, 0)),
        out_shape=jax.ShapeDtypeStruct((p, h), jnp.float32),
        compiler_params=pltpu.CompilerParams(
            dimension_semantics=("parallel",)),
    )(x, bias, w1, w2a, w2b, w3a, w3b, b1p, b2p, b3p)
    return out.reshape(_B, _S1, h)
